# baseline (speedup 1.0000x reference)
import functools

import jax
import jax.numpy as jnp
from jax import lax
from jax.experimental import pallas as pl
from jax.experimental.pallas import tpu as pltpu
from jax.experimental.pallas import tpu_sc as plsc

NUM_WAVEFORMS = 50000
WAVE_LEN = 1024
PARAM_DIM = 8
N_SAMPLES = 16384

NC = 2
NS = 16
NW = NC * NS
B_PER_W = N_SAMPLES // NW
CHUNK = 32
G = B_PER_W // CHUNK


RING = 3


def _waves_body(idx_hbm, plus_hbm, cross_hbm,
                out_plus, out_cross,
                idx_v, wave0_v, wave1_v, wave2_v,
                gsem0, gsem1, gsem2, psem0, psem1, psem2):
    wid = lax.axis_index("s") * NC + lax.axis_index("c")
    base = wid * B_PER_W

    pltpu.sync_copy(idx_hbm.at[wid], idx_v)

    chunks = ([(plus_hbm, out_plus, c) for c in range(G)]
              + [(cross_hbm, out_cross, c) for c in range(G)])
    bufs = (wave0_v, wave1_v, wave2_v)
    gsems = (gsem0, gsem1, gsem2)
    psems = (psem0, psem1, psem2)
    T = len(chunks)

    def _refs(i):
        tab, out, c = chunks[i]
        src = tab.at[idx_v.at[c]]
        dst = out.at[pl.ds(base + c * CHUNK, CHUNK)]
        return src, dst

    def gather_start(i):
        src, _ = _refs(i)
        pltpu.async_copy(src, bufs[i % RING], gsems[i % RING])

    def gather_wait(i):
        src, _ = _refs(i)
        pltpu.make_async_copy(src, bufs[i % RING], gsems[i % RING]).wait()

    def put_start(i):
        _, dst = _refs(i)
        pltpu.async_copy(bufs[i % RING], dst, psems[i % RING])

    def put_wait(i):
        _, dst = _refs(i)
        pltpu.make_async_copy(bufs[i % RING], dst, psems[i % RING]).wait()

    for i in range(RING - 1):
        gather_start(i)
    for i in range(T):
        j = i + RING - 1
        if j < T:
            if j >= RING:
                put_wait(j - RING)
            gather_start(j)
        gather_wait(i)
        put_start(i)
    for i in range(T - RING, T):
        if i >= 0:
            put_wait(i)


def _params_body(idx_hbm, params_hbm, out_params, idx_v, par_v, sem):
    wid = lax.axis_index("s") * NC + lax.axis_index("c")
    base = wid * B_PER_W

    pltpu.sync_copy(idx_hbm.at[wid], idx_v)

    for c in range(G):
        pltpu.async_copy(params_hbm.at[idx_v.at[c]],
                         par_v.at[pl.ds(c * CHUNK, CHUNK)], sem)
    for c in range(G):
        pltpu.make_async_copy(params_hbm.at[idx_v.at[c]],
                              par_v.at[pl.ds(c * CHUNK, CHUNK)], sem).wait()
    pltpu.sync_copy(par_v, out_params.at[pl.ds(base, B_PER_W)])


@jax.jit
def _run(idx, plus, cross, parameters):
    mesh = plsc.VectorSubcoreMesh(core_axis_name="c", subcore_axis_name="s")
    waves_fn = pl.kernel(
        _waves_body,
        out_type=(
            jax.ShapeDtypeStruct((N_SAMPLES, WAVE_LEN), jnp.float32),
            jax.ShapeDtypeStruct((N_SAMPLES, WAVE_LEN), jnp.float32),
        ),
        mesh=mesh,
        scratch_types=[
            pltpu.VMEM((G, CHUNK), jnp.int32),
            pltpu.VMEM((CHUNK, WAVE_LEN), jnp.float32),
            pltpu.VMEM((CHUNK, WAVE_LEN), jnp.float32),
            pltpu.VMEM((CHUNK, WAVE_LEN), jnp.float32),
            pltpu.SemaphoreType.DMA,
            pltpu.SemaphoreType.DMA,
            pltpu.SemaphoreType.DMA,
            pltpu.SemaphoreType.DMA,
            pltpu.SemaphoreType.DMA,
            pltpu.SemaphoreType.DMA,
        ],
    )
    params_fn = pl.kernel(
        _params_body,
        out_type=jax.ShapeDtypeStruct((N_SAMPLES, PARAM_DIM), jnp.float32),
        mesh=mesh,
        scratch_types=[
            pltpu.VMEM((G, CHUNK), jnp.int32),
            pltpu.VMEM((B_PER_W, PARAM_DIM), jnp.float32),
            pltpu.SemaphoreType.DMA,
        ],
        compiler_params=pltpu.CompilerParams(use_tc_tiling_on_sc=False),
    )
    out_plus, out_cross = waves_fn(idx, plus, cross)
    out_params = params_fn(idx, parameters)
    return out_plus, out_cross, out_params


def kernel(N, plus, cross, parameters):
    num_waveforms = plus.shape[0]
    idx = jax.random.randint(jax.random.key(42), (N_SAMPLES,), 0, num_waveforms)
    idx = idx + jnp.asarray(N - N_SAMPLES, dtype=idx.dtype)
    idx = jnp.clip(idx, 0, num_waveforms - 1).astype(jnp.int32)
    idx3 = idx.reshape(NW, G, CHUNK)
    return _run(idx3, plus, cross, parameters)


# speedup vs baseline: 1.0469x; 1.0469x over previous
import functools

import jax
import jax.numpy as jnp
from jax import lax
from jax.experimental import pallas as pl
from jax.experimental.pallas import tpu as pltpu
from jax.experimental.pallas import tpu_sc as plsc

NUM_WAVEFORMS = 50000
WAVE_LEN = 1024
PARAM_DIM = 8
N_SAMPLES = 16384

NC = 2
NS = 16
NW = NC * NS
B_PER_W = N_SAMPLES // NW
CHUNK = 32
G = B_PER_W // CHUNK


RING = 3


def _waves_body(idx_hbm, plus_hbm, cross_hbm,
                out_plus, out_cross,
                idx_v, wave0_v, wave1_v, wave2_v,
                gsem0, gsem1, gsem2, psem0, psem1, psem2):
    wid = lax.axis_index("s") * NC + lax.axis_index("c")
    base = wid * B_PER_W

    pltpu.sync_copy(idx_hbm.at[wid], idx_v)

    chunks = ([(plus_hbm, out_plus, c) for c in range(G)]
              + [(cross_hbm, out_cross, c) for c in range(G)])
    bufs = (wave0_v, wave1_v, wave2_v)
    gsems = (gsem0, gsem1, gsem2)
    psems = (psem0, psem1, psem2)
    T = len(chunks)

    def _refs(i):
        tab, out, c = chunks[i]
        src = tab.at[idx_v.at[c]]
        dst = out.at[pl.ds(base + c * CHUNK, CHUNK)]
        return src, dst

    def gather_start(i):
        src, _ = _refs(i)
        pltpu.async_copy(src, bufs[i % RING], gsems[i % RING])

    def gather_wait(i):
        src, _ = _refs(i)
        pltpu.make_async_copy(src, bufs[i % RING], gsems[i % RING]).wait()

    def put_start(i):
        _, dst = _refs(i)
        pltpu.async_copy(bufs[i % RING], dst, psems[i % RING])

    def put_wait(i):
        _, dst = _refs(i)
        pltpu.make_async_copy(bufs[i % RING], dst, psems[i % RING]).wait()

    for i in range(RING - 1):
        gather_start(i)
    for i in range(T):
        j = i + RING - 1
        if j < T:
            if j >= RING:
                put_wait(j - RING)
            gather_start(j)
        gather_wait(i)
        put_start(i)
    for i in range(T - RING, T):
        if i >= 0:
            put_wait(i)


def _params_body(idx_hbm, params_hbm, dep_hbm, out_params, idx_v, par_v, sem):
    wid = lax.axis_index("s") * NC + lax.axis_index("c")
    base = wid * B_PER_W

    pltpu.sync_copy(idx_hbm.at[wid], idx_v)

    for c in range(G):
        pltpu.async_copy(params_hbm.at[idx_v.at[c]],
                         par_v.at[pl.ds(c * CHUNK, CHUNK)], sem)
    for c in range(G):
        pltpu.make_async_copy(params_hbm.at[idx_v.at[c]],
                              par_v.at[pl.ds(c * CHUNK, CHUNK)], sem).wait()
    pltpu.sync_copy(par_v, out_params.at[pl.ds(base, B_PER_W)])


@jax.jit
def _run(idx, plus, cross, parameters):
    mesh = plsc.VectorSubcoreMesh(core_axis_name="c", subcore_axis_name="s")
    waves_fn = pl.kernel(
        _waves_body,
        out_type=(
            jax.ShapeDtypeStruct((N_SAMPLES, WAVE_LEN), jnp.float32),
            jax.ShapeDtypeStruct((N_SAMPLES, WAVE_LEN), jnp.float32),
        ),
        mesh=mesh,
        scratch_types=[
            pltpu.VMEM((G, CHUNK), jnp.int32),
            pltpu.VMEM((CHUNK, WAVE_LEN), jnp.float32),
            pltpu.VMEM((CHUNK, WAVE_LEN), jnp.float32),
            pltpu.VMEM((CHUNK, WAVE_LEN), jnp.float32),
            pltpu.SemaphoreType.DMA,
            pltpu.SemaphoreType.DMA,
            pltpu.SemaphoreType.DMA,
            pltpu.SemaphoreType.DMA,
            pltpu.SemaphoreType.DMA,
            pltpu.SemaphoreType.DMA,
        ],
    )
    params_fn = pl.kernel(
        _params_body,
        out_type=jax.ShapeDtypeStruct((N_SAMPLES, PARAM_DIM), jnp.float32),
        mesh=mesh,
        scratch_types=[
            pltpu.VMEM((G, CHUNK), jnp.int32),
            pltpu.VMEM((B_PER_W, PARAM_DIM), jnp.float32),
            pltpu.SemaphoreType.DMA,
        ],
        compiler_params=pltpu.CompilerParams(use_tc_tiling_on_sc=False),
    )
    out_plus, out_cross = waves_fn(idx, plus, cross)
    dep = out_plus[:1, :1] * 0.0
    out_params = params_fn(idx, parameters, dep)
    return out_plus, out_cross, out_params


def kernel(N, plus, cross, parameters):
    num_waveforms = plus.shape[0]
    idx = jax.random.randint(jax.random.key(42), (N_SAMPLES,), 0, num_waveforms)
    idx = idx + jnp.asarray(N - N_SAMPLES, dtype=idx.dtype)
    idx = jnp.clip(idx, 0, num_waveforms - 1).astype(jnp.int32)
    idx3 = idx.reshape(NW, G, CHUNK)
    return _run(idx3, plus, cross, parameters)
